# Initial kernel scaffold; baseline (speedup 1.0000x reference)
import jax
import jax.numpy as jnp
from jax.experimental import pallas as pl


def kernel(x, edge_index, edge_weight, Wl0, bl0, Wr0, gamma, beta, running_mean, running_var, Wl1, bl1, Wr1):
    raise NotImplementedError("write your pallas kernel here")


# speedup vs baseline: 3.5962x; 3.5962x over previous
import functools
import jax
import jax.numpy as jnp
from jax import lax
from jax.experimental import pallas as pl
from jax.experimental.pallas import tpu as pltpu
from jax.experimental.pallas import tpu_sc as plsc

_N = 10000
_E = 320000
_IN = 128
_H = 256
_OUT = 128

_NC = 2
_NS = 16
_NW = _NC * _NS
_C = 128
_NCH = 79
_EPT = _NCH * _C
_EPAD = _NW * _EPT
_NA = 10112
_RPT = _NA // _NS
_ROW_CHUNKS = tuple((r0, min(_C, _RPT - r0)) for r0 in range(0, _RPT, _C))
_F = 128


def _sc_feat_body(table, src, dst, ewn, out_s,
                  src_v, dst_v, ew_v, rows_v, acc, gsem):
  c = lax.axis_index("c")
  s = lax.axis_index("s")
  wid = s * _NC + c
  t0 = s * _RPT

  zeros16 = jnp.zeros((16,), jnp.float32)

  def zbody(r, carry):
    for d in range(_F // 16):
      rows_v[r, pl.ds(d * 16, 16)] = zeros16
    return carry
  lax.fori_loop(0, _C, zbody, 0)

  for r0, rn in _ROW_CHUNKS:
    pltpu.sync_copy(rows_v.at[pl.ds(0, rn)], acc.at[pl.ds(t0 + r0, rn)])
  plsc.subcore_barrier()

  def chunk(j, carry):
    base = wid * _EPT + j * _C
    pltpu.sync_copy(src.at[pl.ds(base, _C)], src_v)
    pltpu.sync_copy(dst.at[pl.ds(base, _C)], dst_v.at[0])
    pltpu.sync_copy(ewn.at[pl.ds(base, _C)], ew_v)
    pltpu.async_copy(table.at[src_v], rows_v, gsem).wait()

    def scale(e, cc):
      wv = ew_v[e, :]
      for d in range(_F // 16):
        sl = pl.ds(d * 16, 16)
        rows_v[e, sl] = rows_v[e, sl] * wv
      return cc
    lax.fori_loop(0, _C, scale, 0)

    pltpu.sync_copy(rows_v, acc.at[dst_v.at[0]], add=True)
    return carry
  lax.fori_loop(0, _NCH, chunk, 0)
  plsc.subcore_barrier()

  for r0, rn in _ROW_CHUNKS:
    rb = t0 + r0
    pltpu.sync_copy(acc.at[pl.ds(rb, rn)], out_s.at[c, pl.ds(rb, rn)])


def _make_sc_feat():
  mesh = plsc.VectorSubcoreMesh(core_axis_name="c", subcore_axis_name="s")
  out_type = jax.ShapeDtypeStruct((_NC, _NA, _F), jnp.float32)
  scratch = [
      pltpu.VMEM((_C,), jnp.int32),
      pltpu.VMEM((1, _C), jnp.int32),
      pltpu.VMEM((_C, 16), jnp.float32),
      pltpu.VMEM((_C, _F), jnp.float32),
      pltpu.VMEM_SHARED((_NA, _F), jnp.float32),
      pltpu.SemaphoreType.DMA,
  ]
  return pl.kernel(_sc_feat_body, out_type=out_type, mesh=mesh,
                   scratch_types=scratch)


def _sc_deg_body(dst, out_d, dst_v, ones_v, z16_v, dacc):
  c = lax.axis_index("c")
  s = lax.axis_index("s")
  wid = s * _NC + c
  t0 = s * _RPT

  zeros16 = jnp.zeros((16,), jnp.float32)
  ones16 = jnp.ones((16,), jnp.float32)

  def zbody(r, carry):
    ones_v[r, :] = ones16
    z16_v[r, :] = zeros16
    return carry
  lax.fori_loop(0, _C, zbody, 0)

  for r0, rn in _ROW_CHUNKS:
    pltpu.sync_copy(z16_v.at[pl.ds(0, rn)], dacc.at[pl.ds(t0 + r0, rn)])
  plsc.subcore_barrier()

  def chunk(j, carry):
    base = wid * _EPT + j * _C
    pltpu.sync_copy(dst.at[pl.ds(base, _C)], dst_v.at[0])
    pltpu.sync_copy(ones_v, dacc.at[dst_v.at[0]], add=True)
    return carry
  lax.fori_loop(0, _NCH, chunk, 0)
  plsc.subcore_barrier()

  for r0, rn in _ROW_CHUNKS:
    rb = t0 + r0
    pltpu.sync_copy(dacc.at[pl.ds(rb, rn)], out_d.at[c, pl.ds(rb, rn)])


def _make_sc_deg():
  mesh = plsc.VectorSubcoreMesh(core_axis_name="c", subcore_axis_name="s")
  out_type = jax.ShapeDtypeStruct((_NC, _NA, 16), jnp.float32)
  scratch = [
      pltpu.VMEM((1, _C), jnp.int32),
      pltpu.VMEM((_C, 16), jnp.float32),
      pltpu.VMEM((_C, 16), jnp.float32),
      pltpu.VMEM_SHARED((_NA, 16), jnp.float32),
  ]
  return pl.kernel(_sc_deg_body, out_type=out_type, mesh=mesh,
                   scratch_types=scratch,
                   compiler_params=pltpu.CompilerParams(
                       use_tc_tiling_on_sc=False))


_sc_feat = _make_sc_feat()
_sc_degree = _make_sc_deg()


def _norm_body(w_ref, o_ref):
  w = w_ref[...]
  mn = jnp.min(w)
  mx = jnp.max(w)
  o_ref[...] = jnp.where(mx == mn, jnp.ones_like(w), (w - mn) / (mx - mn))


_norm = pl.pallas_call(
    _norm_body,
    out_shape=jax.ShapeDtypeStruct((_E // 128, 128), jnp.float32))


_BLK = 400
_NBLK = _N // _BLK


def _dense0_body(sp, dp, x, wl0, bl0, wr0, gamma, beta, rm, rv, wl1,
                 h_out, g_out):
  ssum = sp[0] + sp[1]
  dsum = dp[0] + dp[1]
  deg = jnp.clip(dsum[:, 0:1], 1.0, None)
  aggr = ssum / deg
  dn = (((1,), (1,)), ((), ()))
  pre = (lax.dot_general(aggr, wl0[...], dn, preferred_element_type=jnp.float32)
         + bl0[...]
         + lax.dot_general(x[...], wr0[...], dn, preferred_element_type=jnp.float32))
  inv = lax.rsqrt(rv[...] + 1e-5)
  hh = jnp.maximum((pre - rm[...]) * inv * gamma[...] + beta[...], 0.0)
  h_out[...] = hh
  g_out[...] = lax.dot_general(hh, wl1[...], dn, preferred_element_type=jnp.float32)


_dense0 = pl.pallas_call(
    _dense0_body,
    grid=(_NBLK,),
    in_specs=[
        pl.BlockSpec((_NC, _BLK, _F), lambda i: (0, i, 0)),
        pl.BlockSpec((_NC, _BLK, 16), lambda i: (0, i, 0)),
        pl.BlockSpec((_BLK, _IN), lambda i: (i, 0)),
        pl.BlockSpec((_H, _IN), lambda i: (0, 0)),
        pl.BlockSpec((1, _H), lambda i: (0, 0)),
        pl.BlockSpec((_H, _IN), lambda i: (0, 0)),
        pl.BlockSpec((1, _H), lambda i: (0, 0)),
        pl.BlockSpec((1, _H), lambda i: (0, 0)),
        pl.BlockSpec((1, _H), lambda i: (0, 0)),
        pl.BlockSpec((1, _H), lambda i: (0, 0)),
        pl.BlockSpec((_OUT, _H), lambda i: (0, 0)),
    ],
    out_specs=[
        pl.BlockSpec((_BLK, _H), lambda i: (i, 0)),
        pl.BlockSpec((_BLK, _OUT), lambda i: (i, 0)),
    ],
    out_shape=[
        jax.ShapeDtypeStruct((_N, _H), jnp.float32),
        jax.ShapeDtypeStruct((_N, _OUT), jnp.float32),
    ])


def _dense1_body(sp, dp, h, wr1, bl1, lsm_out, var_out, acc_s):
  i = pl.program_id(0)
  ssum = sp[0] + sp[1]
  dsum = dp[0] + dp[1]
  deg = jnp.clip(dsum[:, 0:1], 1.0, None)
  dn = (((1,), (1,)), ((), ()))
  o = (ssum / deg + bl1[...]
       + lax.dot_general(h[...], wr1[...], dn, preferred_element_type=jnp.float32))
  m = jnp.max(o, axis=1, keepdims=True)
  lse = jnp.log(jnp.sum(jnp.exp(o - m), axis=1, keepdims=True)) + m
  lsm_out[...] = o - lse
  bs = jnp.sum(o)
  bss = jnp.sum(o * o)

  @pl.when(i == 0)
  def _():
    acc_s[0] = bs
    acc_s[1] = bss

  @pl.when(i > 0)
  def _():
    acc_s[0] = acc_s[0] + bs
    acc_s[1] = acc_s[1] + bss

  tot = float(_N * _OUT)
  var_out[...] = jnp.full((1, 1), (acc_s[1] - acc_s[0] * acc_s[0] / tot)
                          / (tot - 1.0), jnp.float32)


_dense1 = pl.pallas_call(
    _dense1_body,
    grid=(_NBLK,),
    in_specs=[
        pl.BlockSpec((_NC, _BLK, _OUT), lambda i: (0, i, 0)),
        pl.BlockSpec((_NC, _BLK, 16), lambda i: (0, i, 0)),
        pl.BlockSpec((_BLK, _H), lambda i: (i, 0)),
        pl.BlockSpec((_OUT, _H), lambda i: (0, 0)),
        pl.BlockSpec((1, _OUT), lambda i: (0, 0)),
    ],
    out_specs=[
        pl.BlockSpec((_BLK, _OUT), lambda i: (i, 0)),
        pl.BlockSpec((1, 1), lambda i: (0, 0)),
    ],
    out_shape=[
        jax.ShapeDtypeStruct((_N, _OUT), jnp.float32),
        jax.ShapeDtypeStruct((1, 1), jnp.float32),
    ],
    scratch_shapes=[pltpu.SMEM((2,), jnp.float32)])


def kernel(x, edge_index, edge_weight, Wl0, bl0, Wr0, gamma, beta,
           running_mean, running_var, Wl1, bl1, Wr1):
  ewn = _norm(edge_weight.reshape(_E // 128, 128)).reshape(_E)
  pad = _EPAD - _E
  src = jnp.concatenate([edge_index[0], jnp.zeros((pad,), jnp.int32)])
  dst = jnp.concatenate([edge_index[1], jnp.full((pad,), _N, jnp.int32)])
  ewn_p = jnp.concatenate([ewn, jnp.zeros((pad,), jnp.float32)])
  ewn_r = jnp.broadcast_to(ewn_p[:, None], (_EPAD, 16))

  dp0 = _sc_degree(dst)
  sp0 = _sc_feat(x, src, dst, ewn_r)
  h, g = _dense0(sp0, dp0, x, Wl0, bl0.reshape(1, -1), Wr0,
                 gamma.reshape(1, -1), beta.reshape(1, -1),
                 running_mean.reshape(1, -1), running_var.reshape(1, -1), Wl1)
  sp1 = _sc_feat(g, src, dst, ewn_r)
  lsm, var = _dense1(sp1, dp0, h, Wr1, bl1.reshape(1, -1))
  return lsm, var.reshape(())


# speedup vs baseline: 3.6092x; 1.0036x over previous
import functools
import jax
import jax.numpy as jnp
from jax import lax
from jax.experimental import pallas as pl
from jax.experimental.pallas import tpu as pltpu
from jax.experimental.pallas import tpu_sc as plsc

_N = 10000
_E = 320000
_IN = 128
_H = 256
_OUT = 128

_NC = 2
_NS = 16
_NW = _NC * _NS
_C = 128
_NCH = 79
_EPT = _NCH * _C
_EPAD = _NW * _EPT
_NA = 10112
_RPT = _NA // _NS
_ROW_CHUNKS = tuple((r0, min(_C, _RPT - r0)) for r0 in range(0, _RPT, _C))
_F = 128


def _sc_feat_body(table, src, dst, ewn, out_s,
                  src_v, dst_v, ew_v, rows_v, acc, gsem):
  c = lax.axis_index("c")
  s = lax.axis_index("s")
  wid = s * _NC + c
  t0 = s * _RPT

  zeros16 = jnp.zeros((16,), jnp.float32)

  def zbody(r, carry):
    for d in range(_F // 16):
      rows_v[r, pl.ds(d * 16, 16)] = zeros16
    return carry
  lax.fori_loop(0, _C, zbody, 0)

  for r0, rn in _ROW_CHUNKS:
    pltpu.sync_copy(rows_v.at[pl.ds(0, rn)], acc.at[pl.ds(t0 + r0, rn)])
  plsc.subcore_barrier()

  def chunk(j, carry):
    base = wid * _EPT + j * _C
    pltpu.sync_copy(src.at[pl.ds(base, _C)], src_v)
    pltpu.sync_copy(dst.at[pl.ds(base, _C)], dst_v.at[0])
    pltpu.sync_copy(ewn.at[pl.ds(base, _C)], ew_v)
    pltpu.async_copy(table.at[src_v], rows_v, gsem).wait()

    def scale(e, cc):
      wv = ew_v[e, :]
      for d in range(_F // 16):
        sl = pl.ds(d * 16, 16)
        rows_v[e, sl] = rows_v[e, sl] * wv
      return cc
    lax.fori_loop(0, _C, scale, 0)

    pltpu.sync_copy(rows_v, acc.at[dst_v.at[0]], add=True)
    return carry
  lax.fori_loop(0, _NCH, chunk, 0)
  plsc.subcore_barrier()

  for r0, rn in _ROW_CHUNKS:
    rb = t0 + r0
    pltpu.sync_copy(acc.at[pl.ds(rb, rn)], out_s.at[c, pl.ds(rb, rn)])


def _make_sc_feat():
  mesh = plsc.VectorSubcoreMesh(core_axis_name="c", subcore_axis_name="s")
  out_type = jax.ShapeDtypeStruct((_NC, _NA, _F), jnp.float32)
  scratch = [
      pltpu.VMEM((_C,), jnp.int32),
      pltpu.VMEM((1, _C), jnp.int32),
      pltpu.VMEM((_C, 16), jnp.float32),
      pltpu.VMEM((_C, _F), jnp.float32),
      pltpu.VMEM_SHARED((_NA, _F), jnp.float32),
      pltpu.SemaphoreType.DMA,
  ]
  return pl.kernel(_sc_feat_body, out_type=out_type, mesh=mesh,
                   scratch_types=scratch)


def _sc_deg_body(dst, out_d, dst_v, ones_v, z16_v, dacc):
  c = lax.axis_index("c")
  s = lax.axis_index("s")
  wid = s * _NC + c
  t0 = s * _RPT

  zeros16 = jnp.zeros((16,), jnp.float32)
  ones16 = jnp.ones((16,), jnp.float32)

  def zbody(r, carry):
    ones_v[r, :] = ones16
    z16_v[r, :] = zeros16
    return carry
  lax.fori_loop(0, _C, zbody, 0)

  for r0, rn in _ROW_CHUNKS:
    pltpu.sync_copy(z16_v.at[pl.ds(0, rn)], dacc.at[pl.ds(t0 + r0, rn)])
  plsc.subcore_barrier()

  def chunk(j, carry):
    base = wid * _EPT + j * _C
    pltpu.sync_copy(dst.at[pl.ds(base, _C)], dst_v.at[0])
    pltpu.sync_copy(ones_v, dacc.at[dst_v.at[0]], add=True)
    return carry
  lax.fori_loop(0, _NCH, chunk, 0)
  plsc.subcore_barrier()

  for r0, rn in _ROW_CHUNKS:
    rb = t0 + r0
    pltpu.sync_copy(dacc.at[pl.ds(rb, rn)], out_d.at[c, pl.ds(rb, rn)])


def _make_sc_deg():
  mesh = plsc.VectorSubcoreMesh(core_axis_name="c", subcore_axis_name="s")
  out_type = jax.ShapeDtypeStruct((_NC, _NA, 16), jnp.float32)
  scratch = [
      pltpu.VMEM((1, _C), jnp.int32),
      pltpu.VMEM((_C, 16), jnp.float32),
      pltpu.VMEM((_C, 16), jnp.float32),
      pltpu.VMEM_SHARED((_NA, 16), jnp.float32),
  ]
  return pl.kernel(_sc_deg_body, out_type=out_type, mesh=mesh,
                   scratch_types=scratch,
                   compiler_params=pltpu.CompilerParams(
                       use_tc_tiling_on_sc=False))


_sc_feat = _make_sc_feat()
_sc_degree = _make_sc_deg()


def _norm_body(w_ref, o_ref):
  w = w_ref[...]
  mn = jnp.min(w)
  mx = jnp.max(w)
  o_ref[...] = jnp.where(mx == mn, jnp.ones_like(w), (w - mn) / (mx - mn))


_norm = pl.pallas_call(
    _norm_body,
    out_shape=jax.ShapeDtypeStruct((_E // 128, 128), jnp.float32))


_BLK = 400
_NBLK = _N // _BLK


def _dense0_body(sp, dp, x, wl0, bl0, wr0, gamma, beta, rm, rv, wl1,
                 h_out, g_out):
  ssum = sp[0] + sp[1]
  dsum = dp[0] + dp[1]
  deg = jnp.clip(dsum[:, 0:1], 1.0, None)
  aggr = ssum / deg
  dn = (((1,), (1,)), ((), ()))
  pre = (lax.dot_general(aggr, wl0[...], dn, preferred_element_type=jnp.float32)
         + bl0[...]
         + lax.dot_general(x[...], wr0[...], dn, preferred_element_type=jnp.float32))
  inv = lax.rsqrt(rv[...] + 1e-5)
  hh = jnp.maximum((pre - rm[...]) * inv * gamma[...] + beta[...], 0.0)
  h_out[...] = hh
  g_out[...] = lax.dot_general(hh, wl1[...], dn, preferred_element_type=jnp.float32)


_dense0 = pl.pallas_call(
    _dense0_body,
    grid=(_NBLK,),
    in_specs=[
        pl.BlockSpec((_NC, _BLK, _F), lambda i: (0, i, 0)),
        pl.BlockSpec((_NC, _BLK, 16), lambda i: (0, i, 0)),
        pl.BlockSpec((_BLK, _IN), lambda i: (i, 0)),
        pl.BlockSpec((_H, _IN), lambda i: (0, 0)),
        pl.BlockSpec((1, _H), lambda i: (0, 0)),
        pl.BlockSpec((_H, _IN), lambda i: (0, 0)),
        pl.BlockSpec((1, _H), lambda i: (0, 0)),
        pl.BlockSpec((1, _H), lambda i: (0, 0)),
        pl.BlockSpec((1, _H), lambda i: (0, 0)),
        pl.BlockSpec((1, _H), lambda i: (0, 0)),
        pl.BlockSpec((_OUT, _H), lambda i: (0, 0)),
    ],
    out_specs=[
        pl.BlockSpec((_BLK, _H), lambda i: (i, 0)),
        pl.BlockSpec((_BLK, _OUT), lambda i: (i, 0)),
    ],
    out_shape=[
        jax.ShapeDtypeStruct((_N, _H), jnp.float32),
        jax.ShapeDtypeStruct((_N, _OUT), jnp.float32),
    ])


def _dense1_body(sp, dp, h, wr1, bl1, lsm_out, var_out, acc_s):
  i = pl.program_id(0)
  ssum = sp[0] + sp[1]
  dsum = dp[0] + dp[1]
  deg = jnp.clip(dsum[:, 0:1], 1.0, None)
  dn = (((1,), (1,)), ((), ()))
  o = (ssum / deg + bl1[...]
       + lax.dot_general(h[...], wr1[...], dn, preferred_element_type=jnp.float32))
  m = jnp.max(o, axis=1, keepdims=True)
  lse = jnp.log(jnp.sum(jnp.exp(o - m), axis=1, keepdims=True)) + m
  lsm_out[...] = o - lse
  bs = jnp.sum(o)
  bss = jnp.sum(o * o)

  @pl.when(i == 0)
  def _():
    acc_s[0] = bs
    acc_s[1] = bss

  @pl.when(i > 0)
  def _():
    acc_s[0] = acc_s[0] + bs
    acc_s[1] = acc_s[1] + bss

  tot = float(_N * _OUT)
  var_out[...] = jnp.full((1, 1), (acc_s[1] - acc_s[0] * acc_s[0] / tot)
                          / (tot - 1.0), jnp.float32)


_dense1 = pl.pallas_call(
    _dense1_body,
    grid=(_NBLK,),
    in_specs=[
        pl.BlockSpec((_NC, _BLK, _OUT), lambda i: (0, i, 0)),
        pl.BlockSpec((_NC, _BLK, 16), lambda i: (0, i, 0)),
        pl.BlockSpec((_BLK, _H), lambda i: (i, 0)),
        pl.BlockSpec((_OUT, _H), lambda i: (0, 0)),
        pl.BlockSpec((1, _OUT), lambda i: (0, 0)),
    ],
    out_specs=[
        pl.BlockSpec((_BLK, _OUT), lambda i: (i, 0)),
        pl.BlockSpec((1, 1), lambda i: (0, 0)),
    ],
    out_shape=[
        jax.ShapeDtypeStruct((_N, _OUT), jnp.float32),
        jax.ShapeDtypeStruct((1, 1), jnp.float32),
    ],
    scratch_shapes=[pltpu.SMEM((2,), jnp.float32)])


def kernel(x, edge_index, edge_weight, Wl0, bl0, Wr0, gamma, beta,
           running_mean, running_var, Wl1, bl1, Wr1):
  ewn = _norm(edge_weight.reshape(_E // 128, 128)).reshape(_E)
  pad = _EPAD - _E
  src = jnp.concatenate([edge_index[0], jnp.zeros((pad,), jnp.int32)])
  junk = _N + jnp.arange(pad, dtype=jnp.int32) % (_NA - _N)
  dst = jnp.concatenate([edge_index[1], junk])
  ewn_p = jnp.concatenate([ewn, jnp.zeros((pad,), jnp.float32)])
  ewn_r = jnp.broadcast_to(ewn_p[:, None], (_EPAD, 16))

  dp0 = _sc_degree(dst)
  sp0 = _sc_feat(x, src, dst, ewn_r)
  h, g = _dense0(sp0, dp0, x, Wl0, bl0.reshape(1, -1), Wr0,
                 gamma.reshape(1, -1), beta.reshape(1, -1),
                 running_mean.reshape(1, -1), running_var.reshape(1, -1), Wl1)
  sp1 = _sc_feat(g, src, dst, ewn_r)
  lsm, var = _dense1(sp1, dp0, h, Wr1, bl1.reshape(1, -1))
  return lsm, var.reshape(())
